# baseline (speedup 1.0000x reference)
import jax
import jax.numpy as jnp
from jax.experimental import pallas as pl
from jax.experimental.pallas import tpu as pltpu

N_TOK = 8192
D_MODEL = 4096
N_EXP = 64
TOP_K = 2
TM = 512
NBUF = 4
NSTEP = N_TOK // TM


def _gate_body(x_hbm, wt_ref, b_ref, tw_ref, ti_ref, w_ref, xbuf, sems):
    i = pl.program_id(0)

    H = TM // 2

    def copies(block, slot):
        return [
            pltpu.make_async_copy(
                x_hbm.at[pl.ds(block * TM + h * H, H), :],
                xbuf.at[slot, pl.ds(h * H, H), :],
                sems.at[slot, h])
            for h in range(2)
        ]

    @pl.when(i == 0)
    def _():
        for j in range(NBUF - 1):
            for c in copies(j, j):
                c.start()

    nxt = i + NBUF - 1

    @pl.when(nxt < NSTEP)
    def _():
        for c in copies(nxt, jax.lax.rem(nxt, NBUF)):
            c.start()

    slot = jax.lax.rem(i, NBUF)
    for c in copies(i, slot):
        c.wait()

    logits = jax.lax.dot_general(
        xbuf[slot], wt_ref[...], (((1,), (0,)), ((), ())),
        preferred_element_type=jnp.float32,
        precision=jax.lax.Precision.DEFAULT)
    logits = logits + b_ref[...]
    m = jnp.max(logits, axis=1, keepdims=True)
    e = jnp.exp(logits - m)
    s = jnp.sum(e, axis=1, keepdims=True)
    w = e / s
    w_ref[...] = w
    ids = jax.lax.broadcasted_iota(jnp.int32, (TM, N_EXP), 1)
    m1 = jnp.max(w, axis=1, keepdims=True)
    i1 = jnp.min(jnp.where(w == m1, ids, N_EXP), axis=1, keepdims=True)
    w2 = jnp.where(ids == i1, -1.0, w)
    m2 = jnp.max(w2, axis=1, keepdims=True)
    i2 = jnp.min(jnp.where(w2 == m2, ids, N_EXP), axis=1, keepdims=True)
    tw_ref[...] = jnp.concatenate([m1, m2], axis=1)
    ti_ref[...] = jnp.concatenate([i1, i2], axis=1)


SC_TOK = 512
SC_GRP = 8
SC_TPG = SC_TOK // SC_GRP
KQ = D_MODEL // 4
SC_B = 4


KH = KQ // 2


def _sc_gate_body(x_hbm, wt_hbm, out_hbm, wtq, xb, acc_buf):
    wid = jax.lax.axis_index("s") * 2 + jax.lax.axis_index("c")
    g = wid // 4
    q = jax.lax.rem(wid, 4)
    tok0 = g * SC_TPG

    for kh in range(2):
        k_base = q * KQ + kh * KH
        pltpu.sync_copy(wt_hbm.at[pl.ds(k_base, KH), :], wtq)

        def batch_body(b, _, kh=kh, k_base=k_base):
            pltpu.sync_copy(
                x_hbm.at[pl.ds(tok0 + b * SC_B, SC_B), pl.ds(k_base, KH)], xb)

            def kc_body(kc, accs):
                k0 = kc * 16
                xv = [xb[t, pl.ds(k0, 16)] for t in range(SC_B)]
                new = list(accs)
                for j in range(16):
                    wrow = [wtq[k0 + j, pl.ds(e * 16, 16)] for e in range(4)]
                    for t in range(SC_B):
                        xs = xv[t][j]
                        for e in range(4):
                            new[t * 4 + e] = new[t * 4 + e] + xs * wrow[e]
                return tuple(new)

            accs = jax.lax.fori_loop(
                0, KH // 16, kc_body,
                tuple(jnp.zeros((16,), jnp.float32) for _ in range(SC_B * 4)))
            for t in range(SC_B):
                for e in range(4):
                    row = b * SC_B + t
                    sl = pl.ds(e * 16, 16)
                    if kh == 0:
                        acc_buf[row, sl] = accs[t * 4 + e]
                    else:
                        acc_buf[row, sl] = acc_buf[row, sl] + accs[t * 4 + e]
            return 0

        jax.lax.fori_loop(0, SC_TPG // SC_B, batch_body, 0)

    pltpu.sync_copy(acc_buf, out_hbm.at[q, pl.ds(tok0, SC_TPG), :])


def _sc_probe(x, Wt):
    import jax.experimental.pallas.tpu_sc as plsc
    f = pl.kernel(
        _sc_gate_body,
        out_type=jax.ShapeDtypeStruct((4, SC_TOK, N_EXP), jnp.float32),
        mesh=plsc.VectorSubcoreMesh(core_axis_name="c", subcore_axis_name="s"),
        scratch_types=[
            pltpu.VMEM((KH, N_EXP), jnp.float32),
            pltpu.VMEM((SC_B, KH), jnp.float32),
            pltpu.VMEM((SC_TPG, N_EXP), jnp.float32),
        ],
    )
    return f(x, Wt)


def kernel(x, W, b):
    Wt = W.T
    b2 = b.reshape(1, N_EXP)
    tw, ti, w = pl.pallas_call(
        _gate_body,
        grid=(NSTEP,),
        in_specs=[
            pl.BlockSpec(memory_space=pltpu.HBM),
            pl.BlockSpec((D_MODEL, N_EXP), lambda i: (0, 0)),
            pl.BlockSpec((1, N_EXP), lambda i: (0, 0)),
        ],
        out_specs=[
            pl.BlockSpec((TM, TOP_K), lambda i: (i, 0)),
            pl.BlockSpec((TM, TOP_K), lambda i: (i, 0)),
            pl.BlockSpec((TM, N_EXP), lambda i: (i, 0)),
        ],
        out_shape=[
            jax.ShapeDtypeStruct((N_TOK, TOP_K), jnp.float32),
            jax.ShapeDtypeStruct((N_TOK, TOP_K), jnp.int32),
            jax.ShapeDtypeStruct((N_TOK, N_EXP), jnp.float32),
        ],
        scratch_shapes=[
            pltpu.VMEM((NBUF, TM, D_MODEL), jnp.float32),
            pltpu.SemaphoreType.DMA((NBUF, 2)),
        ],
    )(x, Wt, b2)
    sc_out = _sc_probe(x, Wt)
    del tw, ti, w
    return (sc_out[0, :, :2], sc_out[1, :, :2].astype(jnp.int32), sc_out[2])


# speedup vs baseline: 1.0443x; 1.0443x over previous
import jax
import jax.numpy as jnp
from jax.experimental import pallas as pl
from jax.experimental.pallas import tpu as pltpu

N_TOK = 8192
D_MODEL = 4096
N_EXP = 64
TOP_K = 2
TM = 512
NBUF = 4
NSTEP = N_TOK // TM


def _gate_body(x_hbm, wt_ref, b_ref, tw_ref, ti_ref, w_ref, xbuf, sems):
    i = pl.program_id(0)

    H = TM // 2

    def copies(block, slot):
        return [
            pltpu.make_async_copy(
                x_hbm.at[pl.ds(block * TM + h * H, H), :],
                xbuf.at[slot, pl.ds(h * H, H), :],
                sems.at[slot, h])
            for h in range(2)
        ]

    @pl.when(i == 0)
    def _():
        for j in range(NBUF - 1):
            for c in copies(j, j):
                c.start()

    nxt = i + NBUF - 1

    @pl.when(nxt < NSTEP)
    def _():
        for c in copies(nxt, jax.lax.rem(nxt, NBUF)):
            c.start()

    slot = jax.lax.rem(i, NBUF)
    for c in copies(i, slot):
        c.wait()

    logits = jax.lax.dot_general(
        xbuf[slot], wt_ref[...], (((1,), (0,)), ((), ())),
        preferred_element_type=jnp.float32,
        precision=jax.lax.Precision.DEFAULT)
    logits = logits + b_ref[...]
    m = jnp.max(logits, axis=1, keepdims=True)
    e = jnp.exp(logits - m)
    s = jnp.sum(e, axis=1, keepdims=True)
    w = e / s
    w_ref[...] = w
    ids = jax.lax.broadcasted_iota(jnp.int32, (TM, N_EXP), 1)
    m1 = jnp.max(w, axis=1, keepdims=True)
    i1 = jnp.min(jnp.where(w == m1, ids, N_EXP), axis=1, keepdims=True)
    w2 = jnp.where(ids == i1, -1.0, w)
    m2 = jnp.max(w2, axis=1, keepdims=True)
    i2 = jnp.min(jnp.where(w2 == m2, ids, N_EXP), axis=1, keepdims=True)
    tw_ref[...] = jnp.concatenate([m1, m2], axis=1)
    ti_ref[...] = jnp.concatenate([i1, i2], axis=1)


SC_TOK = 512
SC_GRP = 8
SC_TPG = SC_TOK // SC_GRP
KQ = D_MODEL // 4
SC_B = 4


KH = KQ // 2

_LANE_DNUMS = jax.lax.GatherDimensionNumbers(
    offset_dims=(), collapsed_slice_dims=(0,), start_index_map=(0,))


def _lane_bcast(v, j):
    idx = jnp.full((16, 1), j, dtype=jnp.int32)
    return jax.lax.gather(
        v, idx, _LANE_DNUMS, (1,),
        mode=jax.lax.GatherScatterMode.PROMISE_IN_BOUNDS)


def _sc_gate_body(x_hbm, wt_hbm, out_hbm, wtq, xb, acc_buf):
    wid = jax.lax.axis_index("s") * 2 + jax.lax.axis_index("c")
    g = wid // 4
    q = jax.lax.rem(wid, 4)
    tok0 = g * SC_TPG

    for kh in range(2):
        k_base = q * KQ + kh * KH
        pltpu.sync_copy(wt_hbm.at[pl.ds(k_base, KH), :], wtq)

        def batch_body(b, _, kh=kh, k_base=k_base):
            pltpu.sync_copy(
                x_hbm.at[pl.ds(tok0 + b * SC_B, SC_B), pl.ds(k_base, KH)], xb)

            def kc_body(kc, accs):
                k0 = kc * 16
                xv = [xb[t, pl.ds(k0, 16)] for t in range(SC_B)]
                new = list(accs)
                for j in range(16):
                    wrow = [wtq[k0 + j, pl.ds(e * 16, 16)] for e in range(4)]
                    for t in range(SC_B):
                        xs = _lane_bcast(xv[t], j)
                        for e in range(4):
                            new[t * 4 + e] = new[t * 4 + e] + xs * wrow[e]
                return tuple(new)

            accs = jax.lax.fori_loop(
                0, KH // 16, kc_body,
                tuple(jnp.zeros((16,), jnp.float32) for _ in range(SC_B * 4)))
            for t in range(SC_B):
                for e in range(4):
                    row = b * SC_B + t
                    sl = pl.ds(e * 16, 16)
                    if kh == 0:
                        acc_buf[row, sl] = accs[t * 4 + e]
                    else:
                        acc_buf[row, sl] = acc_buf[row, sl] + accs[t * 4 + e]
            return 0

        jax.lax.fori_loop(0, SC_TPG // SC_B, batch_body, 0)

    pltpu.sync_copy(acc_buf, out_hbm.at[q, pl.ds(tok0, SC_TPG), :])


def _sc_probe(x, Wt):
    import jax.experimental.pallas.tpu_sc as plsc
    f = pl.kernel(
        _sc_gate_body,
        out_type=jax.ShapeDtypeStruct((4, SC_TOK, N_EXP), jnp.float32),
        mesh=plsc.VectorSubcoreMesh(core_axis_name="c", subcore_axis_name="s"),
        scratch_types=[
            pltpu.VMEM((KH, N_EXP), jnp.float32),
            pltpu.VMEM((SC_B, KH), jnp.float32),
            pltpu.VMEM((SC_TPG, N_EXP), jnp.float32),
        ],
    )
    return f(x, Wt)


def kernel(x, W, b):
    Wt = W.T
    b2 = b.reshape(1, N_EXP)
    tw, ti, w = pl.pallas_call(
        _gate_body,
        grid=(NSTEP,),
        in_specs=[
            pl.BlockSpec(memory_space=pltpu.HBM),
            pl.BlockSpec((D_MODEL, N_EXP), lambda i: (0, 0)),
            pl.BlockSpec((1, N_EXP), lambda i: (0, 0)),
        ],
        out_specs=[
            pl.BlockSpec((TM, TOP_K), lambda i: (i, 0)),
            pl.BlockSpec((TM, TOP_K), lambda i: (i, 0)),
            pl.BlockSpec((TM, N_EXP), lambda i: (i, 0)),
        ],
        out_shape=[
            jax.ShapeDtypeStruct((N_TOK, TOP_K), jnp.float32),
            jax.ShapeDtypeStruct((N_TOK, TOP_K), jnp.int32),
            jax.ShapeDtypeStruct((N_TOK, N_EXP), jnp.float32),
        ],
        scratch_shapes=[
            pltpu.VMEM((NBUF, TM, D_MODEL), jnp.float32),
            pltpu.SemaphoreType.DMA((NBUF, 2)),
        ],
    )(x, Wt, b2)
    sc_out = _sc_probe(x, Wt)
    del tw, ti, w
    return (sc_out[0, :, :2], sc_out[1, :, :2].astype(jnp.int32), sc_out[2])


# speedup vs baseline: 1.0447x; 1.0004x over previous
import jax
import jax.numpy as jnp
from jax.experimental import pallas as pl
from jax.experimental.pallas import tpu as pltpu

N_TOK = 8192
D_MODEL = 4096
N_EXP = 64
TOP_K = 2
TM = 512
NBUF = 4
NSTEP = N_TOK // TM


def _gate_body(x_hbm, wt_ref, b_ref, tw_ref, ti_ref, w_ref, xbuf, sems):
    i = pl.program_id(0)

    H = TM // 2

    def copies(block, slot):
        return [
            pltpu.make_async_copy(
                x_hbm.at[pl.ds(block * TM + h * H, H), :],
                xbuf.at[slot, pl.ds(h * H, H), :],
                sems.at[slot, h])
            for h in range(2)
        ]

    @pl.when(i == 0)
    def _():
        for j in range(NBUF - 1):
            for c in copies(j, j):
                c.start()

    nxt = i + NBUF - 1

    @pl.when(nxt < NSTEP)
    def _():
        for c in copies(nxt, jax.lax.rem(nxt, NBUF)):
            c.start()

    slot = jax.lax.rem(i, NBUF)
    for c in copies(i, slot):
        c.wait()

    logits = jax.lax.dot_general(
        xbuf[slot], wt_ref[...], (((1,), (0,)), ((), ())),
        preferred_element_type=jnp.float32,
        precision=jax.lax.Precision.DEFAULT)
    logits = logits + b_ref[...]
    m = jnp.max(logits, axis=1, keepdims=True)
    e = jnp.exp(logits - m)
    s = jnp.sum(e, axis=1, keepdims=True)
    w = e / s
    w_ref[...] = w
    ids = jax.lax.broadcasted_iota(jnp.int32, (TM, N_EXP), 1)
    m1 = jnp.max(w, axis=1, keepdims=True)
    i1 = jnp.min(jnp.where(w == m1, ids, N_EXP), axis=1, keepdims=True)
    w2 = jnp.where(ids == i1, -1.0, w)
    m2 = jnp.max(w2, axis=1, keepdims=True)
    i2 = jnp.min(jnp.where(w2 == m2, ids, N_EXP), axis=1, keepdims=True)
    tw_ref[...] = jnp.concatenate([m1, m2], axis=1)
    ti_ref[...] = jnp.concatenate([i1, i2], axis=1)


SC_TOK = 512
SC_GRP = 8
SC_TPG = SC_TOK // SC_GRP
KQ = D_MODEL // 4
SC_B = 4


KH = KQ // 2

_LANE_DNUMS = jax.lax.GatherDimensionNumbers(
    offset_dims=(), collapsed_slice_dims=(0,), start_index_map=(0,))


def _lane_bcast(v, j):
    idx = jnp.full((16, 1), j, dtype=jnp.int32)
    return jax.lax.gather(
        v, idx, _LANE_DNUMS, (1,),
        mode=jax.lax.GatherScatterMode.PROMISE_IN_BOUNDS)


def _sc_gate_body(x_hbm, wt_hbm, out_hbm, wtq, xb, acc_buf):
    wid = jax.lax.axis_index("s") * 2 + jax.lax.axis_index("c")
    g = wid // 4
    q = jax.lax.rem(wid, 4)
    tok0 = g * SC_TPG

    for kh in range(2):
        k_base = q * KQ + kh * KH
        pltpu.sync_copy(wt_hbm.at[pl.ds(k_base, KH), :], wtq)

        def batch_body(b, _, kh=kh, k_base=k_base):
            pltpu.sync_copy(
                x_hbm.at[pl.ds(tok0 + b * SC_B, SC_B), pl.ds(k_base, KH)], xb)

            def kc_body(kc, accs):
                k0 = kc * 16
                xv = [xb[t, pl.ds(k0, 16)] for t in range(SC_B)]
                new = list(accs)
                for j in range(16):
                    wrow = [wtq[k0 + j, pl.ds(e * 16, 16)] for e in range(4)]
                    for t in range(SC_B):
                        xs = _lane_bcast(xv[t], j)
                        for e in range(4):
                            new[t * 4 + e] = new[t * 4 + e] + xs * wrow[e]
                return tuple(new)

            import jax.experimental.pallas.tpu_sc as plsc
            accs = plsc.parallel_loop(
                0, KH // 16, 1, unroll=2,
                carry=tuple(jnp.zeros((16,), jnp.float32)
                            for _ in range(SC_B * 4)))(kc_body)
            for t in range(SC_B):
                for e in range(4):
                    row = b * SC_B + t
                    sl = pl.ds(e * 16, 16)
                    if kh == 0:
                        acc_buf[row, sl] = accs[t * 4 + e]
                    else:
                        acc_buf[row, sl] = acc_buf[row, sl] + accs[t * 4 + e]
            return 0

        jax.lax.fori_loop(0, SC_TPG // SC_B, batch_body, 0)

    pltpu.sync_copy(acc_buf, out_hbm.at[q, pl.ds(tok0, SC_TPG), :])


def _sc_probe(x, Wt):
    import jax.experimental.pallas.tpu_sc as plsc
    f = pl.kernel(
        _sc_gate_body,
        out_type=jax.ShapeDtypeStruct((4, SC_TOK, N_EXP), jnp.float32),
        mesh=plsc.VectorSubcoreMesh(core_axis_name="c", subcore_axis_name="s"),
        scratch_types=[
            pltpu.VMEM((KH, N_EXP), jnp.float32),
            pltpu.VMEM((SC_B, KH), jnp.float32),
            pltpu.VMEM((SC_TPG, N_EXP), jnp.float32),
        ],
    )
    return f(x, Wt)


def kernel(x, W, b):
    Wt = W.T
    b2 = b.reshape(1, N_EXP)
    tw, ti, w = pl.pallas_call(
        _gate_body,
        grid=(NSTEP,),
        in_specs=[
            pl.BlockSpec(memory_space=pltpu.HBM),
            pl.BlockSpec((D_MODEL, N_EXP), lambda i: (0, 0)),
            pl.BlockSpec((1, N_EXP), lambda i: (0, 0)),
        ],
        out_specs=[
            pl.BlockSpec((TM, TOP_K), lambda i: (i, 0)),
            pl.BlockSpec((TM, TOP_K), lambda i: (i, 0)),
            pl.BlockSpec((TM, N_EXP), lambda i: (i, 0)),
        ],
        out_shape=[
            jax.ShapeDtypeStruct((N_TOK, TOP_K), jnp.float32),
            jax.ShapeDtypeStruct((N_TOK, TOP_K), jnp.int32),
            jax.ShapeDtypeStruct((N_TOK, N_EXP), jnp.float32),
        ],
        scratch_shapes=[
            pltpu.VMEM((NBUF, TM, D_MODEL), jnp.float32),
            pltpu.SemaphoreType.DMA((NBUF, 2)),
        ],
    )(x, Wt, b2)
    sc_out = _sc_probe(x, Wt)
    del tw, ti, w
    return (sc_out[0, :, :2], sc_out[1, :, :2].astype(jnp.int32), sc_out[2])


# speedup vs baseline: 7.2222x; 6.9130x over previous
import jax
import jax.numpy as jnp
from jax.experimental import pallas as pl

N_TOK = 8192
D_MODEL = 4096
N_EXP = 64
TOP_K = 2
TM = 1024


def _gate_body(x_ref, wt_ref, b_ref, tw_ref, ti_ref, w_ref):
    logits = jax.lax.dot_general(
        x_ref[...], wt_ref[...], (((1,), (0,)), ((), ())),
        preferred_element_type=jnp.float32,
        precision=jax.lax.Precision.DEFAULT)
    logits = logits + b_ref[...]
    m = jnp.max(logits, axis=1, keepdims=True)
    e = jnp.exp(logits - m)
    s = jnp.sum(e, axis=1, keepdims=True)
    w = e / s
    w_ref[...] = w
    ids = jax.lax.broadcasted_iota(jnp.int32, (TM, N_EXP), 1)
    m1 = jnp.max(w, axis=1, keepdims=True)
    i1 = jnp.min(jnp.where(w == m1, ids, N_EXP), axis=1, keepdims=True)
    w2 = jnp.where(ids == i1, -1.0, w)
    m2 = jnp.max(w2, axis=1, keepdims=True)
    i2 = jnp.min(jnp.where(w2 == m2, ids, N_EXP), axis=1, keepdims=True)
    tw_ref[...] = jnp.concatenate([m1, m2], axis=1)
    ti_ref[...] = jnp.concatenate([i1, i2], axis=1)


def kernel(x, W, b):
    Wt = W.T
    b2 = b.reshape(1, N_EXP)
    tw, ti, w = pl.pallas_call(
        _gate_body,
        grid=(N_TOK // TM,),
        in_specs=[
            pl.BlockSpec((TM, D_MODEL), lambda i: (i, 0)),
            pl.BlockSpec((D_MODEL, N_EXP), lambda i: (0, 0)),
            pl.BlockSpec((1, N_EXP), lambda i: (0, 0)),
        ],
        out_specs=[
            pl.BlockSpec((TM, TOP_K), lambda i: (i, 0)),
            pl.BlockSpec((TM, TOP_K), lambda i: (i, 0)),
            pl.BlockSpec((TM, N_EXP), lambda i: (i, 0)),
        ],
        out_shape=[
            jax.ShapeDtypeStruct((N_TOK, TOP_K), jnp.float32),
            jax.ShapeDtypeStruct((N_TOK, TOP_K), jnp.int32),
            jax.ShapeDtypeStruct((N_TOK, N_EXP), jnp.float32),
        ],
    )(x, Wt, b2)
    return (tw, ti, w)


# speedup vs baseline: 7.6254x; 1.0558x over previous
import jax
import jax.numpy as jnp
from jax.experimental import pallas as pl

N_TOK = 8192
D_MODEL = 4096
N_EXP = 64
TOP_K = 2
TM = 1024


def _gate_body(x_ref, wt_ref, b_ref, tw_ref, ti_ref, w_ref):
    logits = jax.lax.dot_general(
        x_ref[...], wt_ref[...], (((1,), (1,)), ((), ())),
        preferred_element_type=jnp.float32,
        precision=jax.lax.Precision.DEFAULT)
    logits = logits + b_ref[...]
    m = jnp.max(logits, axis=1, keepdims=True)
    e = jnp.exp(logits - m)
    s = jnp.sum(e, axis=1, keepdims=True)
    w = e / s
    w_ref[...] = w
    ids = jax.lax.broadcasted_iota(jnp.int32, (TM, N_EXP), 1)
    m1 = jnp.max(w, axis=1, keepdims=True)
    i1 = jnp.min(jnp.where(w == m1, ids, N_EXP), axis=1, keepdims=True)
    w2 = jnp.where(ids == i1, -1.0, w)
    m2 = jnp.max(w2, axis=1, keepdims=True)
    i2 = jnp.min(jnp.where(w2 == m2, ids, N_EXP), axis=1, keepdims=True)
    tw_ref[...] = jnp.concatenate([m1, m2], axis=1)
    ti_ref[...] = jnp.concatenate([i1, i2], axis=1)


def kernel(x, W, b):
    b2 = b.reshape(1, N_EXP)
    tw, ti, w = pl.pallas_call(
        _gate_body,
        grid=(N_TOK // TM,),
        in_specs=[
            pl.BlockSpec((TM, D_MODEL), lambda i: (i, 0)),
            pl.BlockSpec((N_EXP, D_MODEL), lambda i: (0, 0)),
            pl.BlockSpec((1, N_EXP), lambda i: (0, 0)),
        ],
        out_specs=[
            pl.BlockSpec((TM, TOP_K), lambda i: (i, 0)),
            pl.BlockSpec((TM, TOP_K), lambda i: (i, 0)),
            pl.BlockSpec((TM, N_EXP), lambda i: (i, 0)),
        ],
        out_shape=[
            jax.ShapeDtypeStruct((N_TOK, TOP_K), jnp.float32),
            jax.ShapeDtypeStruct((N_TOK, TOP_K), jnp.int32),
            jax.ShapeDtypeStruct((N_TOK, N_EXP), jnp.float32),
        ],
    )(x, W, b2)
    return (tw, ti, w)
